# baseline (speedup 1.0000x reference)
import functools

import jax
import jax.numpy as jnp
from jax import lax
from jax.experimental import pallas as pl
from jax.experimental.pallas import tpu as pltpu
from jax.experimental.pallas import tpu_sc as plsc

D = 32
GRP = 128
NBUF = 4
NW = 32
LBLK = 256
BSTRIDE = 249984
LPAD = 250112


def _tc_relayout(tableT):
    nblk = LPAD // LBLK

    def body(t_hbm, out, x_v, sems):
        i = pl.program_id(0)
        for j in range(4):
            pltpu.make_async_copy(
                t_hbm.at[:, pl.ds(j * BSTRIDE + i * LBLK, LBLK)],
                x_v.at[j], sems.at[j],
            ).start()
        for j in range(4):
            pltpu.make_async_copy(
                t_hbm.at[:, pl.ds(0, LBLK)], x_v.at[j], sems.at[j],
            ).wait()
            out[:, j * D:(j + 1) * D] = x_v[j].T

    return pl.pallas_call(
        body,
        grid=(nblk,),
        in_specs=[pl.BlockSpec(memory_space=pl.ANY)],
        out_specs=pl.BlockSpec((LBLK, 4 * D), lambda i: (i, 0)),
        out_shape=jax.ShapeDtypeStruct((LPAD, 4 * D), jnp.float32),
        scratch_shapes=[
            pltpu.VMEM((4, D, LBLK), jnp.float32),
            pltpu.SemaphoreType.DMA((4,)),
        ],
    )(tableT)


def _sc_gather(idx_hi, idx_lo, table4):
    F, B = idx_hi.shape
    BW = B // NW
    NS = BW // GRP
    nsteps = F * NS
    mesh = plsc.VectorSubcoreMesh(core_axis_name="c", subcore_axis_name="s")

    @functools.partial(
        pl.kernel,
        out_type=jax.ShapeDtypeStruct((F, D, B), jnp.float32),
        mesh=mesh,
        compiler_params=pltpu.CompilerParams(
            use_tc_tiling_on_sc=True, needs_layout_passes=False,
            disable_bounds_checks=True),
        scratch_types=[
            pltpu.VMEM((F, BW), jnp.int32),
            pltpu.VMEM((F, BW), jnp.int32),
            pltpu.VMEM((NBUF, GRP, 128), jnp.float32),
            pltpu.VMEM((2, D, GRP), jnp.float32),
            [pltpu.SemaphoreType.DMA] * NBUF,
            [pltpu.SemaphoreType.DMA] * 2,
        ],
    )
    def k(hi_hbm, lo_hbm, table_hbm, out_hbm, hi_v, lo_v, g_v, t_v,
          gsems, ssems):
        wid = lax.axis_index("s") * 2 + lax.axis_index("c")
        b0 = wid * BW
        pltpu.sync_copy(hi_hbm.at[:, pl.ds(b0, BW)], hi_v)
        pltpu.sync_copy(lo_hbm.at[:, pl.ds(b0, BW)], lo_v)

        def fire(s, buf):
            f = s // NS
            c = (s % NS) * GRP
            pltpu.async_copy(
                table_hbm.at[hi_v.at[f, pl.ds(c, GRP)]],
                g_v.at[buf],
                gsems[buf],
            )

        def drain(buf):
            pltpu.make_async_copy(
                table_hbm.at[pl.ds(0, GRP)], g_v.at[buf], gsems[buf],
            ).wait()

        def out_slice(s):
            f = s // NS
            c = (s % NS) * GRP
            return out_hbm.at[f, :, pl.ds(b0 + c, GRP)]

        def extract(s, buf, tb):
            f = s // NS
            c = (s % NS) * GRP
            gb = g_v.at[buf]
            tbb = t_v.at[tb]

            def jbody(jj, carry):
                j0 = jj * 16
                q = lo_v[f, pl.ds(c + j0, 16)]
                row = j0 + lax.iota(jnp.int32, 16)
                for d in range(D):
                    tbb[d, pl.ds(j0, 16)] = plsc.load_gather(gb, [row, q + d])
                return carry

            lax.fori_loop(0, GRP // 16, jbody, 0)

        def store(s, tb):
            pltpu.async_copy(t_v.at[tb], out_slice(s), ssems[tb])

        def wait_store(s, tb):
            pltpu.make_async_copy(t_v.at[tb], out_slice(s), ssems[tb]).wait()

        def step(s, buf, tb, wait_prev, do_fire):
            drain(buf)
            if wait_prev:
                wait_store(s - 2, tb)
            extract(s, buf, tb)
            store(s, tb)
            if do_fire:
                fire(s + NBUF, buf)

        for b in range(NBUF):
            fire(b, b)
        step(0, 0, 0, False, True)
        step(1, 1, 1, False, True)

        def body(i, carry):
            s0 = 4 * i + 2
            for kk in range(4):
                step(s0 + kk, (2 + kk) % NBUF, kk % 2, True, True)
            return carry

        lax.fori_loop(0, (nsteps - 8) // 4, body, 0)
        for s in range(nsteps - 6, nsteps):
            step(s, s % NBUF, s % 2, True, s + NBUF < nsteps)
        wait_store(nsteps - 2, (nsteps - 2) % 2)
        wait_store(nsteps - 1, (nsteps - 1) % 2)

    return k(idx_hi, idx_lo, table4)


def kernel(indices, values_weight):
    B, F = indices.shape
    V, _ = values_weight.shape
    idx = indices.astype(jnp.int32)
    band = jnp.minimum(idx // BSTRIDE, 3)
    idx_hi = (idx - band * BSTRIDE).T
    idx_lo = (band << 5).T
    table4 = _tc_relayout(values_weight.T)
    outP = _sc_gather(idx_hi, idx_lo, table4)
    return outP.transpose(2, 0, 1)


# speedup vs baseline: 1.9263x; 1.9263x over previous
import functools

import jax
import jax.numpy as jnp
from jax import lax
from jax.experimental import pallas as pl
from jax.experimental.pallas import tpu as pltpu
from jax.experimental.pallas import tpu_sc as plsc

D = 32
GRP = 128
NBUF = 4
NW = 32
LBLK = 1024
BSTRIDE = 249728
LPAD = 250880


def _tc_relayout(tableT):
    nblk = LPAD // LBLK

    def body(t_hbm, out, x_v, sems):
        i = pl.program_id(0)
        for j in range(4):
            pltpu.make_async_copy(
                t_hbm.at[:, pl.ds(j * BSTRIDE + i * LBLK, LBLK)],
                x_v.at[j], sems.at[j],
            ).start()
        for j in range(4):
            pltpu.make_async_copy(
                t_hbm.at[:, pl.ds(0, LBLK)], x_v.at[j], sems.at[j],
            ).wait()
            out[:, j * D:(j + 1) * D] = x_v[j].T

    return pl.pallas_call(
        body,
        grid=(nblk,),
        in_specs=[pl.BlockSpec(memory_space=pl.ANY)],
        out_specs=pl.BlockSpec((LBLK, 4 * D), lambda i: (i, 0)),
        out_shape=jax.ShapeDtypeStruct((LPAD, 4 * D), jnp.float32),
        scratch_shapes=[
            pltpu.VMEM((4, D, LBLK), jnp.float32),
            pltpu.SemaphoreType.DMA((4,)),
        ],
    )(tableT)


def _sc_gather(idx_hi, idx_lo, table4):
    F, B = idx_hi.shape
    BW = B // NW
    NS = BW // GRP
    nsteps = F * NS
    mesh = plsc.VectorSubcoreMesh(core_axis_name="c", subcore_axis_name="s")

    @functools.partial(
        pl.kernel,
        out_type=jax.ShapeDtypeStruct((F, D, B), jnp.float32),
        mesh=mesh,
        compiler_params=pltpu.CompilerParams(
            use_tc_tiling_on_sc=True, needs_layout_passes=False,
            disable_bounds_checks=True),
        scratch_types=[
            pltpu.VMEM((F, BW), jnp.int32),
            pltpu.VMEM((F, BW), jnp.int32),
            pltpu.VMEM((NBUF, GRP, 128), jnp.float32),
            pltpu.VMEM((2, D, GRP), jnp.float32),
            [pltpu.SemaphoreType.DMA] * NBUF,
            [pltpu.SemaphoreType.DMA] * 2,
        ],
    )
    def k(hi_hbm, lo_hbm, table_hbm, out_hbm, hi_v, lo_v, g_v, t_v,
          gsems, ssems):
        wid = lax.axis_index("s") * 2 + lax.axis_index("c")
        b0 = wid * BW
        pltpu.sync_copy(hi_hbm.at[:, pl.ds(b0, BW)], hi_v)
        pltpu.sync_copy(lo_hbm.at[:, pl.ds(b0, BW)], lo_v)

        def fire(s, buf):
            f = s // NS
            c = (s % NS) * GRP
            pltpu.async_copy(
                table_hbm.at[hi_v.at[f, pl.ds(c, GRP)]],
                g_v.at[buf],
                gsems[buf],
            )

        def drain(buf):
            pltpu.make_async_copy(
                table_hbm.at[pl.ds(0, GRP)], g_v.at[buf], gsems[buf],
            ).wait()

        def out_slice(s):
            f = s // NS
            c = (s % NS) * GRP
            return out_hbm.at[f, :, pl.ds(b0 + c, GRP)]

        def extract(s, buf, tb):
            f = s // NS
            c = (s % NS) * GRP
            gb = g_v.at[buf]
            tbb = t_v.at[tb]

            def jbody(jj, carry):
                j0 = jj * 16
                q = lo_v[f, pl.ds(c + j0, 16)]
                row = j0 + lax.iota(jnp.int32, 16)
                for d in range(D):
                    tbb[d, pl.ds(j0, 16)] = plsc.load_gather(gb, [row, q + d])
                return carry

            lax.fori_loop(0, GRP // 16, jbody, 0)

        def store(s, tb):
            pltpu.async_copy(t_v.at[tb], out_slice(s), ssems[tb])

        def wait_store(s, tb):
            pltpu.make_async_copy(t_v.at[tb], out_slice(s), ssems[tb]).wait()

        def step(s, buf, tb, wait_prev, do_fire):
            drain(buf)
            if wait_prev:
                wait_store(s - 2, tb)
            extract(s, buf, tb)
            store(s, tb)
            if do_fire:
                fire(s + NBUF, buf)

        for b in range(NBUF):
            fire(b, b)
        step(0, 0, 0, False, True)
        step(1, 1, 1, False, True)

        def body(i, carry):
            s0 = 4 * i + 2
            for kk in range(4):
                step(s0 + kk, (2 + kk) % NBUF, kk % 2, True, True)
            return carry

        lax.fori_loop(0, (nsteps - 8) // 4, body, 0)
        for s in range(nsteps - 6, nsteps):
            step(s, s % NBUF, s % 2, True, s + NBUF < nsteps)
        wait_store(nsteps - 2, (nsteps - 2) % 2)
        wait_store(nsteps - 1, (nsteps - 1) % 2)

    return k(idx_hi, idx_lo, table4)


def kernel(indices, values_weight):
    B, F = indices.shape
    idx = indices.astype(jnp.int32)
    band = jnp.minimum(idx // BSTRIDE, 3)
    idx_hi = (idx - band * BSTRIDE).T
    idx_lo = (band << 5).T
    table4 = _tc_relayout(values_weight.T)
    outP = _sc_gather(idx_hi, idx_lo, table4)
    return outP.transpose(2, 0, 1)
